# baseline (speedup 1.0000x reference)
import jax
import jax.numpy as jnp
from jax import lax
from jax.experimental import pallas as pl
from jax.experimental.pallas import tpu as pltpu
from jax.experimental.pallas import tpu_sc as plsc

_NC = 2
_NS = 16
_LANES = 16
_CHUNK = 512
_UNROLL = 8


def _gru_body(z_hbm, h_hbm, h0_hbm, out_hbm,
              zb, hb, ob, h0b, si0, si1, so0, so1):
    N, D, P = z_hbm.shape
    nw = _NC * _NS
    wid = lax.axis_index("s") * _NC + lax.axis_index("c")
    nchunk = P // _CHUNK
    rows_per_w = N // nw
    sin = (si0, si1)
    sout = (so0, so1)

    def start_in(row, ci, slot):
        off = ci * _CHUNK
        pltpu.async_copy(z_hbm.at[row, :, pl.ds(off, _CHUNK)], zb.at[slot],
                         sin[slot])
        pltpu.async_copy(h_hbm.at[row, :, pl.ds(off, _CHUNK)], hb.at[slot],
                         sin[slot])
        pltpu.async_copy(h0_hbm.at[row, pl.ds(off, _CHUNK)], h0b.at[slot],
                         sin[slot])

    def wait_in(row, ci, slot):
        off = ci * _CHUNK
        pltpu.make_async_copy(z_hbm.at[row, :, pl.ds(off, _CHUNK)],
                              zb.at[slot], sin[slot]).wait()
        pltpu.make_async_copy(h_hbm.at[row, :, pl.ds(off, _CHUNK)],
                              hb.at[slot], sin[slot]).wait()
        pltpu.make_async_copy(h0_hbm.at[row, pl.ds(off, _CHUNK)],
                              h0b.at[slot], sin[slot]).wait()

    def start_out(row, ci, slot):
        off = ci * _CHUNK
        pltpu.async_copy(ob.at[slot], out_hbm.at[row, :, pl.ds(off, _CHUNK)],
                         sout[slot])

    def wait_out(row, ci, slot):
        off = ci * _CHUNK
        pltpu.make_async_copy(ob.at[slot],
                              out_hbm.at[row, :, pl.ds(off, _CHUNK)],
                              sout[slot]).wait()

    def compute(slot):
        def do_group(j, _):
            p0 = j * (_LANES * _UNROLL)
            carry = [h0b[slot, pl.ds(p0 + k * _LANES, _LANES)]
                     for k in range(_UNROLL)]
            for d in range(D - 1, -1, -1):
                for k in range(_UNROLL):
                    pk = p0 + k * _LANES
                    zv = zb[slot, d, pl.ds(pk, _LANES)]
                    hv = hb[slot, d, pl.ds(pk, _LANES)]
                    carry[k] = carry[k] + zv * (hv - carry[k])
                    ob[slot, d, pl.ds(pk, _LANES)] = carry[k]
            return 0

        lax.fori_loop(0, _CHUNK // (_LANES * _UNROLL), do_group, 0)

    def do_row(r, _):
        row = wid * rows_per_w + r
        start_in(row, 0, 0)

        def do_pair(p, _):
            for b in range(2):
                ci = p * 2 + b

                @pl.when(ci + 1 < nchunk)
                def _():
                    start_in(row, ci + 1, 1 - b)

                wait_in(row, ci, b)

                @pl.when(ci >= 2)
                def _():
                    wait_out(row, ci - 2, b)

                compute(b)
                start_out(row, ci, b)
            return 0

        lax.fori_loop(0, nchunk // 2, do_pair, 0)
        wait_out(row, nchunk - 2, 0)
        wait_out(row, nchunk - 1, 1)
        return 0

    lax.fori_loop(0, rows_per_w, do_row, 0)


def kernel(z, _h, h0):
    B, C, D, H, W = z.shape
    N, P = B * C, H * W
    zr = z.reshape(N, D, P)
    hr = _h.reshape(N, D, P)
    h0r = h0.reshape(N, P)
    mesh = plsc.VectorSubcoreMesh(core_axis_name="c", subcore_axis_name="s",
                                  num_cores=_NC, num_subcores=_NS)
    out = pl.kernel(
        _gru_body,
        out_type=jax.ShapeDtypeStruct((N, D, P), jnp.float32),
        mesh=mesh,
        scratch_types=[
            pltpu.VMEM((2, D, _CHUNK), jnp.float32),
            pltpu.VMEM((2, D, _CHUNK), jnp.float32),
            pltpu.VMEM((2, D, _CHUNK), jnp.float32),
            pltpu.VMEM((2, _CHUNK), jnp.float32),
            pltpu.SemaphoreType.DMA,
            pltpu.SemaphoreType.DMA,
            pltpu.SemaphoreType.DMA,
            pltpu.SemaphoreType.DMA,
        ],
    )(zr, hr, h0r)
    return out.reshape(B, C, D, H, W)


# speedup vs baseline: 1.1951x; 1.1951x over previous
import jax
import jax.numpy as jnp
from jax import lax
from jax.experimental import pallas as pl
from jax.experimental.pallas import tpu as pltpu
from jax.experimental.pallas import tpu_sc as plsc

_NC = 2
_NS = 16
_LANES = 16
_CHUNK = 512
_UNROLL = 4


def _gru_body(z_hbm, h_hbm, h0_hbm, out_hbm,
              zb, hb, ob, h0b, si0, si1, so0, so1):
    N, D, P = z_hbm.shape
    nw = _NC * _NS
    wid = lax.axis_index("s") * _NC + lax.axis_index("c")
    nchunk = P // _CHUNK
    rows_per_w = N // nw
    sin = (si0, si1)
    sout = (so0, so1)

    def start_in(row, ci, slot):
        off = ci * _CHUNK
        pltpu.async_copy(z_hbm.at[row, :, pl.ds(off, _CHUNK)], zb.at[slot],
                         sin[slot])
        pltpu.async_copy(h_hbm.at[row, :, pl.ds(off, _CHUNK)], hb.at[slot],
                         sin[slot])
        pltpu.async_copy(h0_hbm.at[row, pl.ds(off, _CHUNK)], h0b.at[slot],
                         sin[slot])

    def wait_in(row, ci, slot):
        off = ci * _CHUNK
        pltpu.make_async_copy(z_hbm.at[row, :, pl.ds(off, _CHUNK)],
                              zb.at[slot], sin[slot]).wait()
        pltpu.make_async_copy(h_hbm.at[row, :, pl.ds(off, _CHUNK)],
                              hb.at[slot], sin[slot]).wait()
        pltpu.make_async_copy(h0_hbm.at[row, pl.ds(off, _CHUNK)],
                              h0b.at[slot], sin[slot]).wait()

    def start_out(row, ci, slot):
        off = ci * _CHUNK
        pltpu.async_copy(ob.at[slot], out_hbm.at[row, :, pl.ds(off, _CHUNK)],
                         sout[slot])

    def wait_out(row, ci, slot):
        off = ci * _CHUNK
        pltpu.make_async_copy(ob.at[slot],
                              out_hbm.at[row, :, pl.ds(off, _CHUNK)],
                              sout[slot]).wait()

    def compute(slot):
        def do_group(j, _):
            p0 = j * (_LANES * _UNROLL)
            carry = [h0b[slot, pl.ds(p0 + k * _LANES, _LANES)]
                     for k in range(_UNROLL)]
            for d in range(D - 1, -1, -1):
                for k in range(_UNROLL):
                    pk = p0 + k * _LANES
                    zv = zb[slot, d, pl.ds(pk, _LANES)]
                    hv = hb[slot, d, pl.ds(pk, _LANES)]
                    carry[k] = carry[k] + zv * (hv - carry[k])
                    ob[slot, d, pl.ds(pk, _LANES)] = carry[k]
            return 0

        lax.fori_loop(0, _CHUNK // (_LANES * _UNROLL), do_group, 0)

    def do_row(r, _):
        row = wid * rows_per_w + r
        start_in(row, 0, 0)

        def do_pair(p, _):
            for b in range(2):
                ci = p * 2 + b

                @pl.when(ci + 1 < nchunk)
                def _():
                    start_in(row, ci + 1, 1 - b)

                wait_in(row, ci, b)

                @pl.when(ci >= 2)
                def _():
                    wait_out(row, ci - 2, b)

                compute(b)
                start_out(row, ci, b)
            return 0

        lax.fori_loop(0, nchunk // 2, do_pair, 0)
        wait_out(row, nchunk - 2, 0)
        wait_out(row, nchunk - 1, 1)
        return 0

    lax.fori_loop(0, rows_per_w, do_row, 0)


def kernel(z, _h, h0):
    B, C, D, H, W = z.shape
    N, P = B * C, H * W
    zr = z.reshape(N, D, P)
    hr = _h.reshape(N, D, P)
    h0r = h0.reshape(N, P)
    mesh = plsc.VectorSubcoreMesh(core_axis_name="c", subcore_axis_name="s",
                                  num_cores=_NC, num_subcores=_NS)
    out = pl.kernel(
        _gru_body,
        out_type=jax.ShapeDtypeStruct((N, D, P), jnp.float32),
        mesh=mesh,
        scratch_types=[
            pltpu.VMEM((2, D, _CHUNK), jnp.float32),
            pltpu.VMEM((2, D, _CHUNK), jnp.float32),
            pltpu.VMEM((2, D, _CHUNK), jnp.float32),
            pltpu.VMEM((2, _CHUNK), jnp.float32),
            pltpu.SemaphoreType.DMA,
            pltpu.SemaphoreType.DMA,
            pltpu.SemaphoreType.DMA,
            pltpu.SemaphoreType.DMA,
        ],
    )(zr, hr, h0r)
    return out.reshape(B, C, D, H, W)


# speedup vs baseline: 1.2085x; 1.0112x over previous
import jax
import jax.numpy as jnp
from jax import lax
from jax.experimental import pallas as pl
from jax.experimental.pallas import tpu as pltpu
from jax.experimental.pallas import tpu_sc as plsc

_NC = 2
_NS = 16
_LANES = 16
_CHUNK = 512
_UNROLL = 2


def _gru_body(z_hbm, h_hbm, h0_hbm, out_hbm,
              zb, hb, ob, h0b, si0, si1, so0, so1):
    N, D, P = z_hbm.shape
    nw = _NC * _NS
    wid = lax.axis_index("s") * _NC + lax.axis_index("c")
    nchunk = P // _CHUNK
    rows_per_w = N // nw
    sin = (si0, si1)
    sout = (so0, so1)

    def start_in(row, ci, slot):
        off = ci * _CHUNK
        pltpu.async_copy(z_hbm.at[row, :, pl.ds(off, _CHUNK)], zb.at[slot],
                         sin[slot])
        pltpu.async_copy(h_hbm.at[row, :, pl.ds(off, _CHUNK)], hb.at[slot],
                         sin[slot])
        pltpu.async_copy(h0_hbm.at[row, pl.ds(off, _CHUNK)], h0b.at[slot],
                         sin[slot])

    def wait_in(row, ci, slot):
        off = ci * _CHUNK
        pltpu.make_async_copy(z_hbm.at[row, :, pl.ds(off, _CHUNK)],
                              zb.at[slot], sin[slot]).wait()
        pltpu.make_async_copy(h_hbm.at[row, :, pl.ds(off, _CHUNK)],
                              hb.at[slot], sin[slot]).wait()
        pltpu.make_async_copy(h0_hbm.at[row, pl.ds(off, _CHUNK)],
                              h0b.at[slot], sin[slot]).wait()

    def start_out(row, ci, slot):
        off = ci * _CHUNK
        pltpu.async_copy(ob.at[slot], out_hbm.at[row, :, pl.ds(off, _CHUNK)],
                         sout[slot])

    def wait_out(row, ci, slot):
        off = ci * _CHUNK
        pltpu.make_async_copy(ob.at[slot],
                              out_hbm.at[row, :, pl.ds(off, _CHUNK)],
                              sout[slot]).wait()

    def compute(slot):
        def do_group(j, _):
            p0 = j * (_LANES * _UNROLL)
            carry = [h0b[slot, pl.ds(p0 + k * _LANES, _LANES)]
                     for k in range(_UNROLL)]
            for d in range(D - 1, -1, -1):
                for k in range(_UNROLL):
                    pk = p0 + k * _LANES
                    zv = zb[slot, d, pl.ds(pk, _LANES)]
                    hv = hb[slot, d, pl.ds(pk, _LANES)]
                    carry[k] = carry[k] + zv * (hv - carry[k])
                    ob[slot, d, pl.ds(pk, _LANES)] = carry[k]
            return 0

        lax.fori_loop(0, _CHUNK // (_LANES * _UNROLL), do_group, 0)

    def do_row(r, _):
        row = wid * rows_per_w + r
        start_in(row, 0, 0)

        def do_pair(p, _):
            for b in range(2):
                ci = p * 2 + b

                @pl.when(ci + 1 < nchunk)
                def _():
                    start_in(row, ci + 1, 1 - b)

                wait_in(row, ci, b)

                @pl.when(ci >= 2)
                def _():
                    wait_out(row, ci - 2, b)

                compute(b)
                start_out(row, ci, b)
            return 0

        lax.fori_loop(0, nchunk // 2, do_pair, 0)
        wait_out(row, nchunk - 2, 0)
        wait_out(row, nchunk - 1, 1)
        return 0

    lax.fori_loop(0, rows_per_w, do_row, 0)


def kernel(z, _h, h0):
    B, C, D, H, W = z.shape
    N, P = B * C, H * W
    zr = z.reshape(N, D, P)
    hr = _h.reshape(N, D, P)
    h0r = h0.reshape(N, P)
    mesh = plsc.VectorSubcoreMesh(core_axis_name="c", subcore_axis_name="s",
                                  num_cores=_NC, num_subcores=_NS)
    out = pl.kernel(
        _gru_body,
        out_type=jax.ShapeDtypeStruct((N, D, P), jnp.float32),
        mesh=mesh,
        scratch_types=[
            pltpu.VMEM((2, D, _CHUNK), jnp.float32),
            pltpu.VMEM((2, D, _CHUNK), jnp.float32),
            pltpu.VMEM((2, D, _CHUNK), jnp.float32),
            pltpu.VMEM((2, _CHUNK), jnp.float32),
            pltpu.SemaphoreType.DMA,
            pltpu.SemaphoreType.DMA,
            pltpu.SemaphoreType.DMA,
            pltpu.SemaphoreType.DMA,
        ],
    )(zr, hr, h0r)
    return out.reshape(B, C, D, H, W)


# speedup vs baseline: 1.3842x; 1.1454x over previous
import jax
import jax.numpy as jnp
from jax import lax
from jax.experimental import pallas as pl
from jax.experimental.pallas import tpu as pltpu
from jax.experimental.pallas import tpu_sc as plsc

_NC = 2
_NS = 16
_LANES = 16
_CHUNK = 512
_UNROLL = 1


def _gru_body(z_hbm, h_hbm, h0_hbm, out_hbm,
              zb, hb, ob, h0b, si0, si1, so0, so1):
    N, D, P = z_hbm.shape
    nw = _NC * _NS
    wid = lax.axis_index("s") * _NC + lax.axis_index("c")
    nchunk = P // _CHUNK
    rows_per_w = N // nw
    sin = (si0, si1)
    sout = (so0, so1)

    def start_in(row, ci, slot):
        off = ci * _CHUNK
        pltpu.async_copy(z_hbm.at[row, :, pl.ds(off, _CHUNK)], zb.at[slot],
                         sin[slot])
        pltpu.async_copy(h_hbm.at[row, :, pl.ds(off, _CHUNK)], hb.at[slot],
                         sin[slot])
        pltpu.async_copy(h0_hbm.at[row, pl.ds(off, _CHUNK)], h0b.at[slot],
                         sin[slot])

    def wait_in(row, ci, slot):
        off = ci * _CHUNK
        pltpu.make_async_copy(z_hbm.at[row, :, pl.ds(off, _CHUNK)],
                              zb.at[slot], sin[slot]).wait()
        pltpu.make_async_copy(h_hbm.at[row, :, pl.ds(off, _CHUNK)],
                              hb.at[slot], sin[slot]).wait()
        pltpu.make_async_copy(h0_hbm.at[row, pl.ds(off, _CHUNK)],
                              h0b.at[slot], sin[slot]).wait()

    def start_out(row, ci, slot):
        off = ci * _CHUNK
        pltpu.async_copy(ob.at[slot], out_hbm.at[row, :, pl.ds(off, _CHUNK)],
                         sout[slot])

    def wait_out(row, ci, slot):
        off = ci * _CHUNK
        pltpu.make_async_copy(ob.at[slot],
                              out_hbm.at[row, :, pl.ds(off, _CHUNK)],
                              sout[slot]).wait()

    def compute(slot):
        def do_group(j, _):
            p0 = j * (_LANES * _UNROLL)
            carry = [h0b[slot, pl.ds(p0 + k * _LANES, _LANES)]
                     for k in range(_UNROLL)]
            for d in range(D - 1, -1, -1):
                for k in range(_UNROLL):
                    pk = p0 + k * _LANES
                    zv = zb[slot, d, pl.ds(pk, _LANES)]
                    hv = hb[slot, d, pl.ds(pk, _LANES)]
                    carry[k] = carry[k] + zv * (hv - carry[k])
                    ob[slot, d, pl.ds(pk, _LANES)] = carry[k]
            return 0

        lax.fori_loop(0, _CHUNK // (_LANES * _UNROLL), do_group, 0)

    def do_row(r, _):
        row = wid * rows_per_w + r
        start_in(row, 0, 0)

        def do_pair(p, _):
            for b in range(2):
                ci = p * 2 + b

                @pl.when(ci + 1 < nchunk)
                def _():
                    start_in(row, ci + 1, 1 - b)

                wait_in(row, ci, b)

                @pl.when(ci >= 2)
                def _():
                    wait_out(row, ci - 2, b)

                compute(b)
                start_out(row, ci, b)
            return 0

        lax.fori_loop(0, nchunk // 2, do_pair, 0)
        wait_out(row, nchunk - 2, 0)
        wait_out(row, nchunk - 1, 1)
        return 0

    lax.fori_loop(0, rows_per_w, do_row, 0)


def kernel(z, _h, h0):
    B, C, D, H, W = z.shape
    N, P = B * C, H * W
    zr = z.reshape(N, D, P)
    hr = _h.reshape(N, D, P)
    h0r = h0.reshape(N, P)
    mesh = plsc.VectorSubcoreMesh(core_axis_name="c", subcore_axis_name="s",
                                  num_cores=_NC, num_subcores=_NS)
    out = pl.kernel(
        _gru_body,
        out_type=jax.ShapeDtypeStruct((N, D, P), jnp.float32),
        mesh=mesh,
        scratch_types=[
            pltpu.VMEM((2, D, _CHUNK), jnp.float32),
            pltpu.VMEM((2, D, _CHUNK), jnp.float32),
            pltpu.VMEM((2, D, _CHUNK), jnp.float32),
            pltpu.VMEM((2, _CHUNK), jnp.float32),
            pltpu.SemaphoreType.DMA,
            pltpu.SemaphoreType.DMA,
            pltpu.SemaphoreType.DMA,
            pltpu.SemaphoreType.DMA,
        ],
    )(zr, hr, h0r)
    return out.reshape(B, C, D, H, W)
